# Initial kernel scaffold; baseline (speedup 1.0000x reference)
import jax
import jax.numpy as jnp
from jax.experimental import pallas as pl


def kernel(x, x_v, W1_w, W1_b, V_w, V_b):
    raise NotImplementedError("write your pallas kernel here")


# speedup vs baseline: 3.6405x; 3.6405x over previous
import functools

import jax
import jax.numpy as jnp
from jax.experimental import pallas as pl
from jax.experimental.pallas import tpu as pltpu

B, N, F, H, K = 4, 4096, 128, 64, 27
NS = 512
RANK_CHUNK = 512
BIG_I32 = 2**30


def _attention_pool_body(x_ref, xv_ref, w1_ref, b1_ref, v_ref, bv_ref,
                         out_ref, xvn_ref, xs_ref, pool_ref, unpool_ref):
    x = x_ref[0]
    xv = xv_ref[0]

    h = jnp.tanh(jnp.dot(x, w1_ref[...], preferred_element_type=jnp.float32)
                 + b1_ref[...])
    s_col = jax.nn.sigmoid(
        jnp.dot(h, v_ref[...], preferred_element_type=jnp.float32)
        + bv_ref[...])
    xs_ref[0] = s_col
    s_row = jnp.transpose(s_col)

    jj = jax.lax.broadcasted_iota(jnp.int32, (RANK_CHUNK, N), 1)
    ranks = jnp.zeros((1, N), jnp.int32)
    for ci in range(N // RANK_CHUNK):
        sc = s_col[ci * RANK_CHUNK:(ci + 1) * RANK_CHUNK]
        ii = jax.lax.broadcasted_iota(jnp.int32, (RANK_CHUNK, N), 0) \
            + ci * RANK_CHUNK
        before = (sc > s_row) | ((sc == s_row) & (ii < jj))
        ranks = ranks + jnp.sum(jnp.where(before, 1, 0), axis=0, keepdims=True)

    rr = jax.lax.broadcasted_iota(jnp.int32, (NS, N), 0)
    msel = jnp.where(ranks == rr, 1.0, 0.0)

    xvT = jnp.transpose(xv)
    kc = [xvT[c:c + 1, :] for c in range(3)]
    qc = [jnp.sum(msel * kc[c], axis=1, keepdims=True) for c in range(3)]
    xq = jnp.concatenate(qc, axis=1)
    xvn_ref[0] = xq

    k2 = (kc[0] * kc[0] + kc[1] * kc[1]) + kc[2] * kc[2]
    q2 = (qc[0] * qc[0] + qc[1] * qc[1]) + qc[2] * qc[2]
    cross = jnp.dot(xq, xvT, preferred_element_type=jnp.float32)
    d = (q2 + k2) - 2.0 * cross

    jjn = jax.lax.broadcasted_iota(jnp.int32, (NS, N), 1)
    lane_k = jax.lax.broadcasted_iota(jnp.int32, (NS, 32), 1)

    def pool_step(k, carry):
        dw, u, pidx = carry
        mval = jnp.min(dw, axis=1, keepdims=True)
        cand = jnp.where(dw == mval, jjn, BIG_I32)
        idx = jnp.min(cand, axis=1, keepdims=True)
        onehot = jjn == idx
        u = u + jnp.where(onehot, 1.0, 0.0)
        dw = jnp.where(onehot, jnp.inf, dw)
        pidx = pidx + jnp.where(lane_k == k, idx, 0)
        return dw, u, pidx

    _, u, pidx = jax.lax.fori_loop(
        0, K, pool_step,
        (d, jnp.zeros((NS, N), jnp.float32), jnp.zeros((NS, 32), jnp.int32)))
    pool_ref[0] = pidx[:, :K]

    a = u * s_row
    denom = jnp.sum(a, axis=1, keepdims=True)
    o = jnp.dot(a, x, preferred_element_type=jnp.float32)
    out_ref[0] = o / denom

    kc_col = [xv[:, c:c + 1] for c in range(3)]
    k2_col = (kc_col[0] * kc_col[0] + kc_col[1] * kc_col[1]) \
        + kc_col[2] * kc_col[2]
    q2_row = jnp.transpose(q2)
    crossT = jnp.dot(xv, jnp.transpose(xq),
                     preferred_element_type=jnp.float32)
    dT = (k2_col + q2_row) - 2.0 * crossT

    jjt = jax.lax.broadcasted_iota(jnp.int32, (N, NS), 1)
    lane_kt = jax.lax.broadcasted_iota(jnp.int32, (N, 32), 1)

    def unpool_step(k, carry):
        dw, pidx = carry
        mval = jnp.min(dw, axis=1, keepdims=True)
        cand = jnp.where(dw == mval, jjt, BIG_I32)
        idx = jnp.min(cand, axis=1, keepdims=True)
        dw = jnp.where(jjt == idx, jnp.inf, dw)
        pidx = pidx + jnp.where(lane_kt == k, idx, 0)
        return dw, pidx

    _, upidx = jax.lax.fori_loop(
        0, K, unpool_step, (dT, jnp.zeros((N, 32), jnp.int32)))
    unpool_ref[0] = upidx[:, :K]


@functools.partial(jax.jit, static_argnames=())
def kernel(x, x_v, W1_w, W1_b, V_w, V_b):
    b1 = W1_b.reshape(1, H)
    bv = V_b.reshape(1, 1)

    full = lambda shape: pl.BlockSpec(shape, lambda b: (0,) * len(shape))
    batched = lambda shape: pl.BlockSpec((1,) + shape,
                                         lambda b: (b,) + (0,) * len(shape))

    out, xvn, xs, pool, unpool = pl.pallas_call(
        _attention_pool_body,
        grid=(B,),
        in_specs=[
            batched((N, F)),
            batched((N, 3)),
            full((F, H)),
            full((1, H)),
            full((H, 1)),
            full((1, 1)),
        ],
        out_specs=[
            batched((NS, F)),
            batched((NS, 3)),
            batched((N, 1)),
            batched((NS, K)),
            batched((N, K)),
        ],
        out_shape=[
            jax.ShapeDtypeStruct((B, NS, F), jnp.float32),
            jax.ShapeDtypeStruct((B, NS, 3), jnp.float32),
            jax.ShapeDtypeStruct((B, N, 1), jnp.float32),
            jax.ShapeDtypeStruct((B, NS, K), jnp.int32),
            jax.ShapeDtypeStruct((B, N, K), jnp.int32),
        ],
        compiler_params=pltpu.CompilerParams(
            dimension_semantics=("arbitrary",)),
    )(x, x_v, W1_w, b1, V_w, bv)
    return out, xvn, xs, pool, unpool


# speedup vs baseline: 3.7364x; 1.0264x over previous
import functools

import jax
import jax.numpy as jnp
from jax.experimental import pallas as pl
from jax.experimental.pallas import tpu as pltpu

B, N, F, H, K = 4, 4096, 128, 64, 27
NS = 512
RANK_CHUNK = 512
BIG_I32 = 2**30


def _attention_pool_body(x_ref, xv_ref, w1_ref, b1_ref, v_ref, bv_ref,
                         out_ref, xvn_ref, xs_ref, pool_ref, unpool_ref):
    x = x_ref[0]
    xv = xv_ref[0]

    h = jnp.tanh(jnp.dot(x, w1_ref[...], preferred_element_type=jnp.float32)
                 + b1_ref[...])
    s_col = jax.nn.sigmoid(
        jnp.dot(h, v_ref[...], preferred_element_type=jnp.float32)
        + bv_ref[...])
    xs_ref[0] = s_col
    s_row = jnp.transpose(s_col)

    jj = jax.lax.broadcasted_iota(jnp.int32, (RANK_CHUNK, N), 1)
    ranks = jnp.zeros((1, N), jnp.int32)
    for ci in range(N // RANK_CHUNK):
        sc = s_col[ci * RANK_CHUNK:(ci + 1) * RANK_CHUNK]
        ii = jax.lax.broadcasted_iota(jnp.int32, (RANK_CHUNK, N), 0) \
            + ci * RANK_CHUNK
        before = (sc > s_row) | ((sc == s_row) & (ii < jj))
        ranks = ranks + jnp.sum(jnp.where(before, 1, 0), axis=0, keepdims=True)

    rr = jax.lax.broadcasted_iota(jnp.int32, (NS, N), 0)
    msel = jnp.where(ranks == rr, 1.0, 0.0)

    xvT = jnp.transpose(xv)
    kc = [xvT[c:c + 1, :] for c in range(3)]
    qc = [jnp.sum(msel * kc[c], axis=1, keepdims=True) for c in range(3)]
    xq = jnp.concatenate(qc, axis=1)
    xvn_ref[0] = xq

    k2 = (kc[0] * kc[0] + kc[1] * kc[1]) + kc[2] * kc[2]
    q2 = (qc[0] * qc[0] + qc[1] * qc[1]) + qc[2] * qc[2]
    cross = jnp.dot(xq, xvT, preferred_element_type=jnp.float32)
    d = (q2 + k2) - 2.0 * cross

    kc_col = [xv[:, c:c + 1] for c in range(3)]
    k2_col = (kc_col[0] * kc_col[0] + kc_col[1] * kc_col[1]) \
        + kc_col[2] * kc_col[2]
    q2_row = jnp.transpose(q2)
    crossT = jnp.dot(xv, jnp.transpose(xq),
                     preferred_element_type=jnp.float32)
    dT = (k2_col + q2_row) - 2.0 * crossT

    jjn = jax.lax.broadcasted_iota(jnp.int32, (NS, N), 1)
    lane_k = jax.lax.broadcasted_iota(jnp.int32, (NS, 32), 1)
    jjt = jax.lax.broadcasted_iota(jnp.int32, (N, NS), 1)
    lane_kt = jax.lax.broadcasted_iota(jnp.int32, (N, 32), 1)

    def knn_step(k, carry):
        dw, u, pidx, dwt, upidx = carry
        mval = jnp.min(dw, axis=1, keepdims=True)
        cand = jnp.where(dw == mval, jjn, BIG_I32)
        idx = jnp.min(cand, axis=1, keepdims=True)
        onehot = jjn == idx
        u = u + jnp.where(onehot, 1.0, 0.0)
        dw = jnp.where(onehot, jnp.inf, dw)
        pidx = pidx + jnp.where(lane_k == k, idx, 0)
        mvalt = jnp.min(dwt, axis=1, keepdims=True)
        candt = jnp.where(dwt == mvalt, jjt, BIG_I32)
        idxt = jnp.min(candt, axis=1, keepdims=True)
        dwt = jnp.where(jjt == idxt, jnp.inf, dwt)
        upidx = upidx + jnp.where(lane_kt == k, idxt, 0)
        return dw, u, pidx, dwt, upidx

    _, u, pidx, _, upidx = jax.lax.fori_loop(
        0, K, knn_step,
        (d, jnp.zeros((NS, N), jnp.float32), jnp.zeros((NS, 32), jnp.int32),
         dT, jnp.zeros((N, 32), jnp.int32)))
    pool_ref[0] = pidx[:, :K]
    unpool_ref[0] = upidx[:, :K]

    a = u * s_row
    denom = jnp.sum(a, axis=1, keepdims=True)
    o = jnp.dot(a, x, preferred_element_type=jnp.float32)
    out_ref[0] = o / denom


@functools.partial(jax.jit, static_argnames=())
def kernel(x, x_v, W1_w, W1_b, V_w, V_b):
    b1 = W1_b.reshape(1, H)
    bv = V_b.reshape(1, 1)

    full = lambda shape: pl.BlockSpec(shape, lambda b: (0,) * len(shape))
    batched = lambda shape: pl.BlockSpec((1,) + shape,
                                         lambda b: (b,) + (0,) * len(shape))

    out, xvn, xs, pool, unpool = pl.pallas_call(
        _attention_pool_body,
        grid=(B,),
        in_specs=[
            batched((N, F)),
            batched((N, 3)),
            full((F, H)),
            full((1, H)),
            full((H, 1)),
            full((1, 1)),
        ],
        out_specs=[
            batched((NS, F)),
            batched((NS, 3)),
            batched((N, 1)),
            batched((NS, K)),
            batched((N, K)),
        ],
        out_shape=[
            jax.ShapeDtypeStruct((B, NS, F), jnp.float32),
            jax.ShapeDtypeStruct((B, NS, 3), jnp.float32),
            jax.ShapeDtypeStruct((B, N, 1), jnp.float32),
            jax.ShapeDtypeStruct((B, NS, K), jnp.int32),
            jax.ShapeDtypeStruct((B, N, K), jnp.int32),
        ],
        compiler_params=pltpu.CompilerParams(
            dimension_semantics=("arbitrary",)),
    )(x, x_v, W1_w, b1, V_w, bv)
    return out, xvn, xs, pool, unpool


# speedup vs baseline: 3.9543x; 1.0583x over previous
import functools

import jax
import jax.numpy as jnp
from jax.experimental import pallas as pl
from jax.experimental.pallas import tpu as pltpu

B, N, F, H, K = 4, 4096, 128, 64, 27
NS = 512
RANK_CHUNK = 512
BIG_I32 = 2**30


def _attention_pool_body(x_ref, xv_ref, w1_ref, b1_ref, v_ref, bv_ref,
                         out_ref, xvn_ref, xs_ref, pool_ref, unpool_ref):
    x = x_ref[0]
    xv = xv_ref[0]

    h = jnp.tanh(jnp.dot(x, w1_ref[...], preferred_element_type=jnp.float32)
                 + b1_ref[...])
    s_col = jax.nn.sigmoid(
        jnp.dot(h, v_ref[...], preferred_element_type=jnp.float32)
        + bv_ref[...])
    xs_ref[0] = s_col
    s_row = jnp.transpose(s_col)

    jj = jax.lax.broadcasted_iota(jnp.int32, (RANK_CHUNK, N), 1)
    ranks = jnp.zeros((1, N), jnp.int32)
    for ci in range(N // RANK_CHUNK):
        sc = s_col[ci * RANK_CHUNK:(ci + 1) * RANK_CHUNK]
        ii = jax.lax.broadcasted_iota(jnp.int32, (RANK_CHUNK, N), 0) \
            + ci * RANK_CHUNK
        before = (sc > s_row) | ((sc == s_row) & (ii < jj))
        ranks = ranks + jnp.sum(jnp.where(before, 1, 0), axis=0, keepdims=True)

    rr = jax.lax.broadcasted_iota(jnp.int32, (NS, N), 0)
    msel = jnp.where(ranks == rr, 1.0, 0.0)

    xvT = jnp.transpose(xv)
    kc = [xvT[c:c + 1, :] for c in range(3)]
    qc = [jnp.sum(msel * kc[c], axis=1, keepdims=True) for c in range(3)]
    xq = jnp.concatenate(qc, axis=1)
    xvn_ref[0] = xq

    k2 = (kc[0] * kc[0] + kc[1] * kc[1]) + kc[2] * kc[2]
    q2 = (qc[0] * qc[0] + qc[1] * qc[1]) + qc[2] * qc[2]
    cross = jnp.dot(xq, xvT, preferred_element_type=jnp.float32)
    d = (q2 + k2) - 2.0 * cross

    kc_col = [xv[:, c:c + 1] for c in range(3)]
    k2_col = (kc_col[0] * kc_col[0] + kc_col[1] * kc_col[1]) \
        + kc_col[2] * kc_col[2]
    q2_row = jnp.transpose(q2)
    crossT = jnp.dot(xv, jnp.transpose(xq),
                     preferred_element_type=jnp.float32)
    dT = (k2_col + q2_row) - 2.0 * crossT

    jjn = jax.lax.broadcasted_iota(jnp.int32, (NS, N), 1)
    lane_k = jax.lax.broadcasted_iota(jnp.int32, (NS, 32), 1)
    jjt = jax.lax.broadcasted_iota(jnp.int32, (N, NS), 1)
    lane_kt = jax.lax.broadcasted_iota(jnp.int32, (N, 32), 1)

    def knn_step(k, carry):
        dw, u, pidx, dwt, upidx = carry
        mval = jnp.min(dw, axis=1, keepdims=True)
        cand = jnp.where(dw == mval, jjn, BIG_I32)
        idx = jnp.min(cand, axis=1, keepdims=True)
        onehot = jjn == idx
        u = jnp.where(onehot, 1.0, u)
        dw = jnp.where(onehot, jnp.inf, dw)
        pidx = pidx + jnp.where(lane_k == k, idx, 0)
        mvalt = jnp.min(dwt, axis=1, keepdims=True)
        candt = jnp.where(dwt == mvalt, jjt, BIG_I32)
        idxt = jnp.min(candt, axis=1, keepdims=True)
        dwt = jnp.where(jjt == idxt, jnp.inf, dwt)
        upidx = upidx + jnp.where(lane_kt == k, idxt, 0)
        return dw, u, pidx, dwt, upidx

    _, u, pidx, _, upidx = jax.lax.fori_loop(
        0, K, knn_step,
        (d, jnp.zeros((NS, N), jnp.float32), jnp.zeros((NS, 32), jnp.int32),
         dT, jnp.zeros((N, 32), jnp.int32)))
    pool_ref[0] = pidx[:, :K]
    unpool_ref[0] = upidx[:, :K]

    a = u * s_row
    denom = jnp.sum(a, axis=1, keepdims=True)
    o = jnp.dot(a, x, preferred_element_type=jnp.float32)
    out_ref[0] = o / denom


@functools.partial(jax.jit, static_argnames=())
def kernel(x, x_v, W1_w, W1_b, V_w, V_b):
    b1 = W1_b.reshape(1, H)
    bv = V_b.reshape(1, 1)

    full = lambda shape: pl.BlockSpec(shape, lambda b: (0,) * len(shape))
    batched = lambda shape: pl.BlockSpec((1,) + shape,
                                         lambda b: (b,) + (0,) * len(shape))

    out, xvn, xs, pool, unpool = pl.pallas_call(
        _attention_pool_body,
        grid=(B,),
        in_specs=[
            batched((N, F)),
            batched((N, 3)),
            full((F, H)),
            full((1, H)),
            full((H, 1)),
            full((1, 1)),
        ],
        out_specs=[
            batched((NS, F)),
            batched((NS, 3)),
            batched((N, 1)),
            batched((NS, K)),
            batched((N, K)),
        ],
        out_shape=[
            jax.ShapeDtypeStruct((B, NS, F), jnp.float32),
            jax.ShapeDtypeStruct((B, NS, 3), jnp.float32),
            jax.ShapeDtypeStruct((B, N, 1), jnp.float32),
            jax.ShapeDtypeStruct((B, NS, K), jnp.int32),
            jax.ShapeDtypeStruct((B, N, K), jnp.int32),
        ],
        compiler_params=pltpu.CompilerParams(
            dimension_semantics=("arbitrary",)),
    )(x, x_v, W1_w, b1, V_w, bv)
    return out, xvn, xs, pool, unpool


# speedup vs baseline: 4.6033x; 1.1641x over previous
import functools

import jax
import jax.numpy as jnp
from jax import lax
from jax.experimental import pallas as pl
from jax.experimental.pallas import tpu as pltpu
from jax.experimental.pallas import tpu_sc as plsc

B, N, F, H, K = 4, 4096, 128, 64, 27
NS = 512
RANK_CHUNK = 512
BIG_I32 = 2**30

NW = 32
QPT = (B * NS) // NW
CQ = 4
CR = CQ * K
NCHUNK = QPT // CQ


def _prep_body(x_ref, xv_ref, w1_ref, b1_ref, v_ref, bv_ref,
               xs_ref, xs16_ref, xvn_ref, d_ref, dt_ref):
    x = x_ref[0]
    xv = xv_ref[0]

    h = jnp.tanh(jnp.dot(x, w1_ref[...], preferred_element_type=jnp.float32)
                 + b1_ref[...])
    s_col = jax.nn.sigmoid(
        jnp.dot(h, v_ref[...], preferred_element_type=jnp.float32)
        + bv_ref[...])
    xs_ref[0] = s_col
    xs16_ref[0] = jnp.broadcast_to(s_col, (N, 128))
    s_row = jnp.transpose(s_col)

    jj = lax.broadcasted_iota(jnp.int32, (RANK_CHUNK, N), 1)
    ranks = jnp.zeros((1, N), jnp.int32)
    for ci in range(N // RANK_CHUNK):
        sc = s_col[ci * RANK_CHUNK:(ci + 1) * RANK_CHUNK]
        ii = lax.broadcasted_iota(jnp.int32, (RANK_CHUNK, N), 0) \
            + ci * RANK_CHUNK
        before = (sc > s_row) | ((sc == s_row) & (ii < jj))
        ranks = ranks + jnp.sum(jnp.where(before, 1, 0), axis=0, keepdims=True)

    rr = lax.broadcasted_iota(jnp.int32, (NS, N), 0)
    msel = jnp.where(ranks == rr, 1.0, 0.0)

    xvT = jnp.transpose(xv)
    kc = [xvT[c:c + 1, :] for c in range(3)]
    qc = [jnp.sum(msel * kc[c], axis=1, keepdims=True) for c in range(3)]
    xq = jnp.concatenate(qc, axis=1)
    xvn_ref[0] = xq

    k2 = (kc[0] * kc[0] + kc[1] * kc[1]) + kc[2] * kc[2]
    q2 = (qc[0] * qc[0] + qc[1] * qc[1]) + qc[2] * qc[2]
    cross = jnp.dot(xq, xvT, preferred_element_type=jnp.float32)
    d_ref[0] = (q2 + k2) - 2.0 * cross

    kc_col = [xv[:, c:c + 1] for c in range(3)]
    k2_col = (kc_col[0] * kc_col[0] + kc_col[1] * kc_col[1]) \
        + kc_col[2] * kc_col[2]
    q2_row = jnp.transpose(q2)
    crossT = jnp.dot(xv, jnp.transpose(xq),
                     preferred_element_type=jnp.float32)
    dt_ref[0] = (k2_col + q2_row) - 2.0 * crossT


def _pool_body(d_ref, pool_ref, pg_ref):
    dw0 = d_ref[0]
    jjn = lax.broadcasted_iota(jnp.int32, (NS, N), 1)
    lane_k = lax.broadcasted_iota(jnp.int32, (NS, 32), 1)

    def step(k, carry):
        dw, pidx = carry
        mval = jnp.min(dw, axis=1, keepdims=True)
        cand = jnp.where(dw == mval, jjn, BIG_I32)
        idx = jnp.min(cand, axis=1, keepdims=True)
        dw = jnp.where(jjn == idx, jnp.inf, dw)
        pidx = pidx + jnp.where(lane_k == k, idx, 0)
        return dw, pidx

    _, pidx = lax.fori_loop(0, K, step,
                            (dw0, jnp.zeros((NS, 32), jnp.int32)))
    pool_ref[0] = pidx[:, :K]
    pg_ref[0] = pidx[:, :K] + pl.program_id(0) * N


def _unpool_body(dt_ref, unpool_ref):
    dw0 = dt_ref[0]
    jjt = lax.broadcasted_iota(jnp.int32, (N, NS), 1)
    lane_k = lax.broadcasted_iota(jnp.int32, (N, 32), 1)

    def step(k, carry):
        dw, pidx = carry
        mval = jnp.min(dw, axis=1, keepdims=True)
        cand = jnp.where(dw == mval, jjt, BIG_I32)
        idx = jnp.min(cand, axis=1, keepdims=True)
        dw = jnp.where(jjt == idx, jnp.inf, dw)
        pidx = pidx + jnp.where(lane_k == k, idx, 0)
        return dw, pidx

    _, pidx = lax.fori_loop(0, K, step,
                            (dw0, jnp.zeros((N, 32), jnp.int32)))
    unpool_ref[0] = pidx[:, :K]


def _sc_attn_body(x_hbm, xs16_hbm, pg_hbm, out_hbm,
                  idx_v, rows_v, srows_v, ob_v, sem):
    wid = lax.axis_index("s") * 2 + lax.axis_index("c")
    qbase = wid * QPT

    pltpu.sync_copy(pg_hbm.at[wid], idx_v)

    def chunk(c, carry):
        cpa = pltpu.async_copy(x_hbm.at[idx_v.at[c]], rows_v, sem)
        cpb = pltpu.async_copy(xs16_hbm.at[idx_v.at[c]], srows_v, sem)
        cpa.wait()
        cpb.wait()
        for qq in range(CQ):
            sv = [srows_v[qq * K + k, pl.ds(0, 16)] for k in range(K)]
            ssum = sv[0]
            for k in range(1, K):
                ssum = ssum + sv[k]
            inv = 1.0 / ssum
            wk = [s * inv for s in sv]
            for seg in range(F // 16):
                acc = wk[0] * rows_v[qq * K, pl.ds(seg * 16, 16)]
                for k in range(1, K):
                    acc = acc + wk[k] * rows_v[qq * K + k, pl.ds(seg * 16, 16)]
                ob_v[qq, pl.ds(seg * 16, 16)] = acc
        pltpu.sync_copy(ob_v, out_hbm.at[pl.ds(qbase + c * CQ, CQ)])
        return carry

    lax.fori_loop(0, NCHUNK, chunk, 0)


_sc_attn = functools.partial(
    pl.kernel,
    out_type=jax.ShapeDtypeStruct((B * NS, F), jnp.float32),
    mesh=plsc.VectorSubcoreMesh(core_axis_name="c", subcore_axis_name="s"),
    scratch_types=[
        pltpu.VMEM((NCHUNK, CR), jnp.int32),
        pltpu.VMEM((CR, F), jnp.float32),
        pltpu.VMEM((CR, 128), jnp.float32),
        pltpu.VMEM((CQ, F), jnp.float32),
        pltpu.SemaphoreType.DMA,
    ],
)(_sc_attn_body)


@jax.jit
def kernel(x, x_v, W1_w, W1_b, V_w, V_b):
    b1 = W1_b.reshape(1, H)
    bv = V_b.reshape(1, 1)

    full = lambda shape: pl.BlockSpec(shape, lambda b: (0,) * len(shape))
    batched = lambda shape: pl.BlockSpec((1,) + shape,
                                         lambda b: (b,) + (0,) * len(shape))
    params = pltpu.CompilerParams(dimension_semantics=("arbitrary",))

    xs, xs16, xvn, d, dt = pl.pallas_call(
        _prep_body,
        grid=(B,),
        in_specs=[batched((N, F)), batched((N, 3)), full((F, H)),
                  full((1, H)), full((H, 1)), full((1, 1))],
        out_specs=[batched((N, 1)), batched((N, 128)), batched((NS, 3)),
                   batched((NS, N)), batched((N, NS))],
        out_shape=[
            jax.ShapeDtypeStruct((B, N, 1), jnp.float32),
            jax.ShapeDtypeStruct((B, N, 128), jnp.float32),
            jax.ShapeDtypeStruct((B, NS, 3), jnp.float32),
            jax.ShapeDtypeStruct((B, NS, N), jnp.float32),
            jax.ShapeDtypeStruct((B, N, NS), jnp.float32),
        ],
        compiler_params=params,
    )(x, x_v, W1_w, b1, V_w, bv)

    pool, pg = pl.pallas_call(
        _pool_body,
        grid=(B,),
        in_specs=[batched((NS, N))],
        out_specs=[batched((NS, K)), batched((NS, K))],
        out_shape=[jax.ShapeDtypeStruct((B, NS, K), jnp.int32),
                   jax.ShapeDtypeStruct((B, NS, K), jnp.int32)],
        compiler_params=params,
    )(d)

    (unpool,) = pl.pallas_call(
        _unpool_body,
        grid=(B,),
        in_specs=[batched((N, NS))],
        out_specs=[batched((N, K))],
        out_shape=[jax.ShapeDtypeStruct((B, N, K), jnp.int32)],
        compiler_params=params,
    )(dt)

    out = _sc_attn(x.reshape(B * N, F),
                   xs16.reshape(B * N, 128),
                   pg.reshape(NW, NCHUNK, CR))
    return out.reshape(B, NS, F), xvn, xs, pool, unpool
